# baseline (speedup 1.0000x reference)
import jax
import jax.numpy as jnp
from jax.experimental import pallas as pl
from jax.experimental.pallas import tpu as pltpu

_OP_EQ, _OP_LT, _OP_GT = 0, 1, 2
_ALPHA, _BETA = 1.2, 0.7
_B, _D = 8192, 2048
_BM = 256
_NB = _B // _BM
_L = 128
_TR = 64
_NK = _D // _L
_NT = _BM // _TR


def _loss_body(op_full_ref, op_ref, j1_ref, j2_ref, out_ref, acc_ref,
               stats_ref):
    i = pl.program_id(0)

    @pl.when(i == 0)
    def _init():
        opf = op_full_ref[0, :]
        stats_ref[0] = jnp.sum((opf == _OP_EQ).astype(jnp.float32))
        stats_ref[1] = jnp.sum((opf == _OP_LT).astype(jnp.float32))
        stats_ref[2] = jnp.sum((opf == _OP_GT).astype(jnp.float32))
        acc_ref[...] = jnp.zeros((3, _TR, _L), jnp.float32)

    ones_b = jnp.ones((_L, _L), dtype=jnp.bfloat16)
    acc0 = acc_ref[0]
    acc1 = acc_ref[1]
    acc2 = acc_ref[2]
    for r in range(_NT):
        r0 = r * _TR
        a = j1_ref[r0:r0 + _TR, 0:_L].astype(jnp.bfloat16)
        b = j2_ref[r0:r0 + _TR, 0:_L].astype(jnp.bfloat16)
        p1 = a * a
        p2 = b * b
        p12 = a * b
        for k in range(1, _NK):
            c0 = k * _L
            a = j1_ref[r0:r0 + _TR, c0:c0 + _L].astype(jnp.bfloat16)
            b = j2_ref[r0:r0 + _TR, c0:c0 + _L].astype(jnp.bfloat16)
            p1 += a * a
            p2 += b * b
            p12 += a * b
        s1 = jax.lax.dot(p1, ones_b, preferred_element_type=jnp.float32)
        s2 = jax.lax.dot(p2, ones_b, preferred_element_type=jnp.float32)
        pd = (p1 + p2 - 2.0 * p12).astype(jnp.float32)
        op_t = op_ref[r0:r0 + _TR, :]
        eq = (op_t == _OP_EQ).astype(jnp.float32)
        dn = jnp.sqrt(s1) - jnp.sqrt(s2)
        acc0 = acc0 + eq * pd
        acc1 = acc1 + jnp.maximum(dn, 0.0)
        acc2 = acc2 + jnp.maximum(-dn, 0.0)
    acc_ref[0] = acc0
    acc_ref[1] = acc1
    acc_ref[2] = acc2

    @pl.when(i == _NB - 1)
    def _finalize():
        inv_l = 1.0 / _L
        eq_sd = jnp.sum(acc_ref[0])
        lt_sum = jnp.sum(acc_ref[1]) * inv_l
        gt_sum = jnp.sum(acc_ref[2]) * inv_l
        eq_cnt = stats_ref[0]
        has_lt = (stats_ref[1] > 0.0).astype(jnp.float32)
        has_gt = (stats_ref[2] > 0.0).astype(jnp.float32)
        eq_loss = eq_sd / jnp.maximum(eq_cnt * _D, 1.0)
        lt_loss = lt_sum * (1.0 / _B)
        gt_loss = gt_sum * (1.0 / _B)
        out_ref[0, 0] = (_ALPHA * eq_loss
                         + _BETA * (has_lt * lt_loss + has_gt * gt_loss))


def kernel(joint1_embedding, joint2_embedding, operation):
    op_row = operation.reshape(1, _B)
    out = pl.pallas_call(
        _loss_body,
        grid=(_NB,),
        in_specs=[
            pl.BlockSpec((1, _B), lambda i: (0, 0)),
            pl.BlockSpec((_BM, 1), lambda i: (i, 0)),
            pl.BlockSpec((_BM, _D), lambda i: (i, 0)),
            pl.BlockSpec((_BM, _D), lambda i: (i, 0)),
        ],
        out_specs=pl.BlockSpec(memory_space=pltpu.SMEM),
        out_shape=jax.ShapeDtypeStruct((1, 1), jnp.float32),
        scratch_shapes=[
            pltpu.VMEM((3, _TR, _L), jnp.float32),
            pltpu.SMEM((3,), jnp.float32),
        ],
    )(op_row, operation, joint1_embedding, joint2_embedding)
    return out[0, 0]


# speedup vs baseline: 1.0760x; 1.0760x over previous
import jax
import jax.numpy as jnp
from jax.experimental import pallas as pl
from jax.experimental.pallas import tpu as pltpu

_OP_EQ, _OP_LT, _OP_GT = 0, 1, 2
_ALPHA, _BETA = 1.2, 0.7
_B, _D = 8192, 2048
_BM = 512
_NB = _B // _BM
_L = 128
_TR = 64
_NT = _BM // _TR
_NS = 4
_CS = _D // _NS
_NKS = _CS // _L


def _loss_body(op_full_ref, op_ref, j1s0, j1s1, j1s2, j1s3,
               j2s0, j2s1, j2s2, j2s3, out_ref, acc_ref, stats_ref):
    i = pl.program_id(0)

    @pl.when(i == 0)
    def _init():
        opf = op_full_ref[0, :]
        stats_ref[0] = jnp.sum((opf == _OP_EQ).astype(jnp.float32))
        stats_ref[1] = jnp.sum((opf == _OP_LT).astype(jnp.float32))
        stats_ref[2] = jnp.sum((opf == _OP_GT).astype(jnp.float32))
        acc_ref[...] = jnp.zeros((3, _TR, _L), jnp.float32)

    j1_streams = (j1s0, j1s1, j1s2, j1s3)
    j2_streams = (j2s0, j2s1, j2s2, j2s3)
    ones_b = jnp.ones((_L, _L), dtype=jnp.bfloat16)
    acc0 = acc_ref[0]
    acc1 = acc_ref[1]
    acc2 = acc_ref[2]
    for r in range(_NT):
        r0 = r * _TR
        p1 = None
        for s in range(_NS):
            ja_ref = j1_streams[s]
            jb_ref = j2_streams[s]
            for k in range(_NKS):
                c0 = k * _L
                a = ja_ref[r0:r0 + _TR, c0:c0 + _L].astype(jnp.bfloat16)
                b = jb_ref[r0:r0 + _TR, c0:c0 + _L].astype(jnp.bfloat16)
                if p1 is None:
                    p1 = a * a
                    p2 = b * b
                    p12 = a * b
                else:
                    p1 += a * a
                    p2 += b * b
                    p12 += a * b
        s1 = jax.lax.dot(p1, ones_b, preferred_element_type=jnp.float32)
        s2 = jax.lax.dot(p2, ones_b, preferred_element_type=jnp.float32)
        pd = (p1 + p2 - 2.0 * p12).astype(jnp.float32)
        op_t = op_ref[r0:r0 + _TR, :]
        eq = (op_t == _OP_EQ).astype(jnp.float32)
        dn = jnp.sqrt(s1) - jnp.sqrt(s2)
        acc0 = acc0 + eq * pd
        acc1 = acc1 + jnp.maximum(dn, 0.0)
        acc2 = acc2 + jnp.maximum(-dn, 0.0)
    acc_ref[0] = acc0
    acc_ref[1] = acc1
    acc_ref[2] = acc2

    @pl.when(i == _NB - 1)
    def _finalize():
        inv_l = 1.0 / _L
        eq_sd = jnp.sum(acc_ref[0])
        lt_sum = jnp.sum(acc_ref[1]) * inv_l
        gt_sum = jnp.sum(acc_ref[2]) * inv_l
        eq_cnt = stats_ref[0]
        has_lt = (stats_ref[1] > 0.0).astype(jnp.float32)
        has_gt = (stats_ref[2] > 0.0).astype(jnp.float32)
        eq_loss = eq_sd / jnp.maximum(eq_cnt * _D, 1.0)
        lt_loss = lt_sum * (1.0 / _B)
        gt_loss = gt_sum * (1.0 / _B)
        out_ref[0, 0] = (_ALPHA * eq_loss
                         + _BETA * (has_lt * lt_loss + has_gt * gt_loss))


def kernel(joint1_embedding, joint2_embedding, operation):
    op_row = operation.reshape(1, _B)
    col_specs = [
        pl.BlockSpec((_BM, _CS), lambda i, s=s: (i, s)) for s in range(_NS)
    ]
    out = pl.pallas_call(
        _loss_body,
        grid=(_NB,),
        in_specs=[
            pl.BlockSpec((1, _B), lambda i: (0, 0)),
            pl.BlockSpec((_BM, 1), lambda i: (i, 0)),
        ] + col_specs + col_specs,
        out_specs=pl.BlockSpec(memory_space=pltpu.SMEM),
        out_shape=jax.ShapeDtypeStruct((1, 1), jnp.float32),
        scratch_shapes=[
            pltpu.VMEM((3, _TR, _L), jnp.float32),
            pltpu.SMEM((3,), jnp.float32),
        ],
    )(op_row, operation,
      joint1_embedding, joint1_embedding, joint1_embedding, joint1_embedding,
      joint2_embedding, joint2_embedding, joint2_embedding, joint2_embedding)
    return out[0, 0]
